# baseline (speedup 1.0000x reference)
import functools

import jax
import jax.numpy as jnp
from jax import lax
from jax.experimental import pallas as pl
from jax.experimental.pallas import tpu as pltpu
from jax.experimental.pallas import tpu_sc as plsc

NPTS = 512
L = 16
WAYS = 4


def _fps_sc_body(nsc, pos_hbm, out_hbm, xv, yv, zv, dist, outv, stage, pbuf, shared):
    n = xv.shape[0]
    quarter = dist.shape[0]
    s = lax.axis_index("s")
    c = lax.axis_index("c")
    g = s // WAYS
    m_id = lax.rem(s, WAYS)
    b = c * (nsc // 2) + g
    base = m_id * quarter

    pltpu.sync_copy(pos_hbm.at[pl.ds(pl.multiple_of((0 * nsc + b) * n, n), n)], xv)
    pltpu.sync_copy(pos_hbm.at[pl.ds(pl.multiple_of((1 * nsc + b) * n, n), n)], yv)
    pltpu.sync_copy(pos_hbm.at[pl.ds(pl.multiple_of((2 * nsc + b) * n, n), n)], zv)

    lanes = lax.iota(jnp.int32, L)

    @plsc.parallel_loop(0, quarter, L, unroll=8)
    def init_j(j):
        dist[pl.ds(pl.multiple_of(j, L), L)] = jnp.full((L,), 1e10, jnp.float32)

    def iter_i(t, carry):
        f, picks = carry
        fv = jnp.full((L,), f, jnp.int32)
        picks = jnp.where(lanes == t, fv, picks)
        cx = plsc.load_gather(xv, [fv])
        cy = plsc.load_gather(yv, [fv])
        cz = plsc.load_gather(zv, [fv])

        carry0 = tuple(
            (jnp.full((L,), -1.0, jnp.float32), jnp.zeros((L,), jnp.int32))
            for _ in range(8)
        )

        @plsc.parallel_loop(0, quarter, 8 * L, unroll=1, carry=carry0)
        def chunk(j, acc):
            new = []
            for k in range(8):
                mx, mi = acc[k]
                off = j + k * L
                ds_l = pl.ds(pl.multiple_of(off, L), L)
                ds_g = pl.ds(pl.multiple_of(base + off, L), L)
                dx = xv[ds_g] - cx
                dy = yv[ds_g] - cy
                dz = zv[ds_g] - cz
                d2 = dx * dx + (dy * dy + dz * dz)
                dv = jnp.minimum(dist[ds_l], d2)
                dist[ds_l] = dv
                gt = dv > mx
                new.append(
                    (
                        jnp.where(gt, dv, mx),
                        jnp.where(gt, jnp.full((L,), off, jnp.int32), mi),
                    )
                )
            return tuple(new)

        def mrg(a, bb):
            va, ia = a
            vb, ib = bb
            take_b = (vb > va) | ((vb == va) & (ib < ia))
            return jnp.where(take_b, vb, va), jnp.where(take_b, ib, ia)

        pairs = list(chunk)
        while len(pairs) > 1:
            pairs = [
                mrg(pairs[i], pairs[i + 1]) for i in range(0, len(pairs), 2)
            ]
        mx, mib = pairs[0]
        mi = jnp.full((L,), base, jnp.int32) + mib + lanes
        m = jnp.max(mx)
        cand = jnp.where(mx == m, mi, jnp.int32(n))
        li = jnp.min(cand)

        mvec = jnp.full((L,), m, jnp.float32)
        stage[...] = jnp.where(
            lanes == 0,
            plsc.bitcast(mvec, jnp.int32),
            jnp.full((L,), li, jnp.int32),
        )
        pltpu.sync_copy(stage, shared.at[pl.ds(pl.multiple_of(s * L, L), L)])
        plsc.subcore_barrier()
        pltpu.sync_copy(
            shared.at[pl.ds(pl.multiple_of(g * (WAYS * L), WAYS * L), WAYS * L)],
            pbuf,
        )
        bv = jnp.float32(-1.0)
        bi = jnp.int32(0)
        for k in range(WAYS):
            pv = pbuf[pl.ds(k * L, L)]
            pm = plsc.bitcast(pv, jnp.float32)[0]
            pi = pv[1]
            take = (pm > bv) | ((pm == bv) & (pi < bi))
            bv = jnp.where(take, pm, bv)
            bi = jnp.where(take, pi, bi)
        return bi, picks

    def outer_o(o, f):
        f, picks = lax.fori_loop(0, L, iter_i, (f, jnp.zeros((L,), jnp.int32)))
        outv[pl.ds(pl.multiple_of(o * L, L), L)] = picks
        return f

    lax.fori_loop(0, NPTS // L, outer_o, jnp.int32(0))

    @pl.when(m_id == 0)
    def _():
        pltpu.sync_copy(outv, out_hbm.at[pl.ds(pl.multiple_of(b * NPTS, NPTS), NPTS)])


def _fps_tc_body(xr, yr, zr, outr, dist):
    K, n = xr.shape
    ch = 128
    nch = n // ch
    lane_io = lax.broadcasted_iota(jnp.int32, (K, ch), 1)

    def init_j(j, carry):
        dist[:, pl.ds(pl.multiple_of(j * ch, ch), ch)] = jnp.full(
            (K, ch), 1e10, jnp.float32
        )
        return carry

    lax.fori_loop(0, nch, init_j, 0)

    def iter_t(t, carry):
        fi, cx, cy, cz, picks = carry
        picks = jnp.where(lane_io == t, fi, picks)

        def chunk(j, mc):
            rmx, rmb, rcx, rcy, rcz = mc
            ds_ = pl.ds(pl.multiple_of(j * ch, ch), ch)
            xc = xr[:, ds_]
            yc = yr[:, ds_]
            zc = zr[:, ds_]
            dx = xc - cx
            dy = yc - cy
            dz = zc - cz
            d2 = dx * dx + (dy * dy + dz * dz)
            dv = jnp.minimum(dist[:, ds_], d2)
            dist[:, ds_] = dv
            gt = dv > rmx
            return (
                jnp.where(gt, dv, rmx),
                jnp.where(gt, jnp.full((K, ch), j * ch, jnp.int32), rmb),
                jnp.where(gt, xc, rcx),
                jnp.where(gt, yc, rcy),
                jnp.where(gt, zc, rcz),
            )

        z32 = jnp.zeros((K, ch), jnp.int32)
        zf = jnp.zeros((K, ch), jnp.float32)
        rmx, rmb, rcx, rcy, rcz = lax.fori_loop(
            0,
            nch,
            chunk,
            (jnp.full((K, ch), -1.0, jnp.float32), z32, zf, zf, zf),
        )
        idxm = rmb + lane_io
        m = jnp.max(rmx, axis=1, keepdims=True)
        ci = jnp.where(rmx == m, idxm, jnp.int32(n))
        fi_new = jnp.min(ci, axis=1, keepdims=True)
        w = ci == fi_new
        cx_n = jnp.sum(jnp.where(w, rcx, 0.0), axis=1, keepdims=True)
        cy_n = jnp.sum(jnp.where(w, rcy, 0.0), axis=1, keepdims=True)
        cz_n = jnp.sum(jnp.where(w, rcz, 0.0), axis=1, keepdims=True)
        return fi_new, cx_n, cy_n, cz_n, picks

    def outer_t(o, carry):
        fi, cx, cy, cz = carry
        fi, cx, cy, cz, picks = lax.fori_loop(
            0,
            ch,
            iter_t,
            (fi, cx, cy, cz, jnp.zeros((K, ch), jnp.int32)),
        )
        outr[:, pl.ds(pl.multiple_of(o * ch, ch), ch)] = picks
        return fi, cx, cy, cz

    fi0 = jnp.zeros((K, 1), jnp.int32)
    lax.fori_loop(
        0, NPTS // ch, outer_t, (fi0, xr[:, 0:1], yr[:, 0:1], zr[:, 0:1])
    )


def kernel(pos):
    B, N, C = pos.shape
    nsc = B // 2
    ntc = B - nsc
    pos_flat = jnp.transpose(pos[:nsc], (2, 0, 1)).reshape(3 * nsc * N)
    mesh = plsc.VectorSubcoreMesh(core_axis_name="c", subcore_axis_name="s")
    sc_fps = pl.kernel(
        functools.partial(_fps_sc_body, nsc),
        mesh=mesh,
        compiler_params=pltpu.CompilerParams(needs_layout_passes=False),
        out_type=jax.ShapeDtypeStruct((nsc * NPTS,), jnp.int32),
        scratch_types=[
            pltpu.VMEM((N,), jnp.float32),
            pltpu.VMEM((N,), jnp.float32),
            pltpu.VMEM((N,), jnp.float32),
            pltpu.VMEM((N // WAYS,), jnp.float32),
            pltpu.VMEM((NPTS,), jnp.int32),
            pltpu.VMEM((L,), jnp.int32),
            pltpu.VMEM((WAYS * L,), jnp.int32),
            pltpu.VMEM_SHARED((16 * L,), jnp.int32),
        ],
    )
    sc_out = sc_fps(pos_flat).reshape(nsc, NPTS)

    xt = pos[nsc:, :, 0]
    yt = pos[nsc:, :, 1]
    zt = pos[nsc:, :, 2]
    tc_out = pl.pallas_call(
        _fps_tc_body,
        out_shape=jax.ShapeDtypeStruct((ntc, NPTS), jnp.int32),
        scratch_shapes=[pltpu.VMEM((ntc, N), jnp.float32)],
    )(xt, yt, zt)

    return jnp.concatenate([sc_out, tc_out], axis=0)


# speedup vs baseline: 8.5866x; 8.5866x over previous
import functools

import jax
import jax.numpy as jnp
from jax import lax
from jax.experimental import pallas as pl
from jax.experimental.pallas import tpu as pltpu
from jax.experimental.pallas import tpu_sc as plsc

NPTS = 512
L = 16
WAYS = 4


def _fps_sc_body(nsc, pos_hbm, out_hbm, xv, yv, zv, dist, outv, stage, pbuf, shared):
    n = xv.shape[0]
    quarter = dist.shape[0]
    s = lax.axis_index("s")
    c = lax.axis_index("c")
    g = s // WAYS
    m_id = lax.rem(s, WAYS)
    b = c * (nsc // 2) + g
    base = m_id * quarter

    pltpu.sync_copy(pos_hbm.at[pl.ds(pl.multiple_of((0 * nsc + b) * n, n), n)], xv)
    pltpu.sync_copy(pos_hbm.at[pl.ds(pl.multiple_of((1 * nsc + b) * n, n), n)], yv)
    pltpu.sync_copy(pos_hbm.at[pl.ds(pl.multiple_of((2 * nsc + b) * n, n), n)], zv)

    lanes = lax.iota(jnp.int32, L)

    @plsc.parallel_loop(0, quarter, L, unroll=8)
    def init_j(j):
        dist[pl.ds(pl.multiple_of(j, L), L)] = jnp.full((L,), 1e10, jnp.float32)

    def iter_i(t, carry):
        f, picks = carry
        fv = jnp.full((L,), f, jnp.int32)
        picks = jnp.where(lanes == t, fv, picks)
        cx = plsc.load_gather(xv, [fv])
        cy = plsc.load_gather(yv, [fv])
        cz = plsc.load_gather(zv, [fv])

        carry0 = tuple(
            (jnp.full((L,), -1.0, jnp.float32), jnp.zeros((L,), jnp.int32))
            for _ in range(8)
        )

        @plsc.parallel_loop(0, quarter, 8 * L, unroll=1, carry=carry0)
        def chunk(j, acc):
            new = []
            for k in range(8):
                mx, mi = acc[k]
                off = j + k * L
                ds_l = pl.ds(pl.multiple_of(off, L), L)
                ds_g = pl.ds(pl.multiple_of(base + off, L), L)
                dx = xv[ds_g] - cx
                dy = yv[ds_g] - cy
                dz = zv[ds_g] - cz
                d2 = dx * dx + (dy * dy + dz * dz)
                dv = jnp.minimum(dist[ds_l], d2)
                dist[ds_l] = dv
                gt = dv > mx
                new.append(
                    (
                        jnp.where(gt, dv, mx),
                        jnp.where(gt, jnp.full((L,), off, jnp.int32), mi),
                    )
                )
            return tuple(new)

        def mrg(a, bb):
            va, ia = a
            vb, ib = bb
            take_b = (vb > va) | ((vb == va) & (ib < ia))
            return jnp.where(take_b, vb, va), jnp.where(take_b, ib, ia)

        pairs = list(chunk)
        while len(pairs) > 1:
            pairs = [
                mrg(pairs[i], pairs[i + 1]) for i in range(0, len(pairs), 2)
            ]
        mx, mib = pairs[0]
        mi = jnp.full((L,), base, jnp.int32) + mib + lanes
        m = jnp.max(mx)
        cand = jnp.where(mx == m, mi, jnp.int32(n))
        li = jnp.min(cand)

        mvec = jnp.full((L,), m, jnp.float32)
        stage[...] = jnp.where(
            lanes == 0,
            plsc.bitcast(mvec, jnp.int32),
            jnp.full((L,), li, jnp.int32),
        )
        pltpu.sync_copy(stage, shared.at[pl.ds(pl.multiple_of(s * L, L), L)])
        plsc.subcore_barrier()
        pltpu.sync_copy(
            shared.at[pl.ds(pl.multiple_of(g * (WAYS * L), WAYS * L), WAYS * L)],
            pbuf,
        )
        bv = jnp.float32(-1.0)
        bi = jnp.int32(0)
        for k in range(WAYS):
            pv = pbuf[pl.ds(k * L, L)]
            pm = plsc.bitcast(pv, jnp.float32)[0]
            pi = pv[1]
            take = (pm > bv) | ((pm == bv) & (pi < bi))
            bv = jnp.where(take, pm, bv)
            bi = jnp.where(take, pi, bi)
        return bi, picks

    def outer_o(o, f):
        f, picks = lax.fori_loop(0, L, iter_i, (f, jnp.zeros((L,), jnp.int32)))
        outv[pl.ds(pl.multiple_of(o * L, L), L)] = picks
        return f

    lax.fori_loop(0, NPTS // L, outer_o, jnp.int32(0))

    @pl.when(m_id == 0)
    def _():
        pltpu.sync_copy(outv, out_hbm.at[pl.ds(pl.multiple_of(b * NPTS, NPTS), NPTS)])


def _fps_tc_body(xr, yr, zr, outr, dist):
    K, n = xr.shape
    ch = 128
    nch = n // ch
    lane_io = lax.broadcasted_iota(jnp.int32, (K, ch), 1)

    def init_j(j, carry):
        dist[:, pl.ds(pl.multiple_of(j * ch, ch), ch)] = jnp.full(
            (K, ch), 1e10, jnp.float32
        )
        return carry

    lax.fori_loop(0, nch, init_j, 0)

    def iter_t(t, carry):
        fi, cx, cy, cz, picks = carry
        picks = jnp.where(lane_io == t, fi, picks)
        zf = jnp.zeros((K, ch), jnp.float32)
        cxb = cx + zf
        cyb = cy + zf
        czb = cz + zf
        unr = 8

        def chunk(j, acc):
            new = []
            for k in range(unr):
                rmx, rmb, rcx, rcy, rcz = acc[k]
                jj = j * (unr * ch) + k * ch
                ds_ = pl.ds(pl.multiple_of(jj, ch), ch)
                xc = xr[:, ds_]
                yc = yr[:, ds_]
                zc = zr[:, ds_]
                dx = xc - cxb
                dy = yc - cyb
                dz = zc - czb
                d2 = dx * dx + (dy * dy + dz * dz)
                dv = jnp.minimum(dist[:, ds_], d2)
                dist[:, ds_] = dv
                gt = dv > rmx
                new.append(
                    (
                        jnp.where(gt, dv, rmx),
                        jnp.where(gt, jnp.full((K, ch), jj, jnp.int32), rmb),
                        jnp.where(gt, xc, rcx),
                        jnp.where(gt, yc, rcy),
                        jnp.where(gt, zc, rcz),
                    )
                )
            return tuple(new)

        z32 = jnp.zeros((K, ch), jnp.int32)
        acc0 = tuple(
            (jnp.full((K, ch), -1.0, jnp.float32), z32, zf, zf, zf)
            for _ in range(unr)
        )
        accs = lax.fori_loop(0, nch // unr, chunk, acc0)

        def mrg(a, bb):
            take_b = (bb[0] > a[0]) | ((bb[0] == a[0]) & (bb[1] < a[1]))
            return tuple(jnp.where(take_b, y, x) for x, y in zip(a, bb))

        pairs = list(accs)
        while len(pairs) > 1:
            pairs = [
                mrg(pairs[i], pairs[i + 1]) for i in range(0, len(pairs), 2)
            ]
        rmx, rmb, rcx, rcy, rcz = pairs[0]
        idxm = rmb + lane_io
        m = jnp.max(rmx, axis=1, keepdims=True)
        ci = jnp.where(rmx == m, idxm, jnp.int32(n))
        fi_new = jnp.min(ci, axis=1, keepdims=True)
        w = ci == fi_new
        cx_n = jnp.sum(jnp.where(w, rcx, 0.0), axis=1, keepdims=True)
        cy_n = jnp.sum(jnp.where(w, rcy, 0.0), axis=1, keepdims=True)
        cz_n = jnp.sum(jnp.where(w, rcz, 0.0), axis=1, keepdims=True)
        return fi_new, cx_n, cy_n, cz_n, picks

    def outer_t(o, carry):
        fi, cx, cy, cz = carry
        fi, cx, cy, cz, picks = lax.fori_loop(
            0,
            ch,
            iter_t,
            (fi, cx, cy, cz, jnp.zeros((K, ch), jnp.int32)),
        )
        outr[:, pl.ds(pl.multiple_of(o * ch, ch), ch)] = picks
        return fi, cx, cy, cz

    fi0 = jnp.zeros((K, 1), jnp.int32)
    lax.fori_loop(
        0, NPTS // ch, outer_t, (fi0, xr[:, 0:1], yr[:, 0:1], zr[:, 0:1])
    )


def kernel(pos):
    B, N, C = pos.shape
    nsc = B // 2
    ntc = B - nsc
    pos_flat = jnp.transpose(pos[:nsc], (2, 0, 1)).reshape(3 * nsc * N)
    mesh = plsc.VectorSubcoreMesh(core_axis_name="c", subcore_axis_name="s")
    sc_fps = pl.kernel(
        functools.partial(_fps_sc_body, nsc),
        mesh=mesh,
        compiler_params=pltpu.CompilerParams(needs_layout_passes=False),
        out_type=jax.ShapeDtypeStruct((nsc * NPTS,), jnp.int32),
        scratch_types=[
            pltpu.VMEM((N,), jnp.float32),
            pltpu.VMEM((N,), jnp.float32),
            pltpu.VMEM((N,), jnp.float32),
            pltpu.VMEM((N // WAYS,), jnp.float32),
            pltpu.VMEM((NPTS,), jnp.int32),
            pltpu.VMEM((L,), jnp.int32),
            pltpu.VMEM((WAYS * L,), jnp.int32),
            pltpu.VMEM_SHARED((16 * L,), jnp.int32),
        ],
    )
    sc_out = sc_fps(pos_flat).reshape(nsc, NPTS)

    xt = pos[nsc:, :, 0]
    yt = pos[nsc:, :, 1]
    zt = pos[nsc:, :, 2]
    tc_out = pl.pallas_call(
        _fps_tc_body,
        out_shape=jax.ShapeDtypeStruct((ntc, NPTS), jnp.int32),
        scratch_shapes=[pltpu.VMEM((ntc, N), jnp.float32)],
    )(xt, yt, zt)

    return jnp.concatenate([sc_out, tc_out], axis=0)
